# baseline (speedup 1.0000x reference)
import functools

import jax
import jax.numpy as jnp
from jax import lax
from jax.experimental import pallas as pl
from jax.experimental.pallas import tpu as pltpu
from jax.experimental.pallas import tpu_sc as plsc

_EMBED_DIM = 128
_N_EXPERTS = 64

_CHUNK = 128


def _make_gather(n_rows, embed_dim, n_workers, dtype):
    rows_per_w = n_rows // n_workers
    chunks_per_w = rows_per_w // _CHUNK
    mesh = plsc.VectorSubcoreMesh(core_axis_name="c", subcore_axis_name="s")
    num_cores = plsc.get_sparse_core_info().num_cores

    @functools.partial(
        pl.kernel,
        mesh=mesh,
        out_type=jax.ShapeDtypeStruct((n_rows, embed_dim), dtype),
        scratch_types=[
            pltpu.VMEM((chunks_per_w, _CHUNK), jnp.int32),
            pltpu.VMEM((rows_per_w, embed_dim), dtype),
        ]
        + [pltpu.SemaphoreType.DMA] * chunks_per_w
        + [pltpu.SemaphoreType.DMA],
    )
    def gather_kernel(idx_hbm, table_hbm, out_hbm, idx_v, rows_v, *sems):
        gsems, osem = sems[:-1], sems[-1]
        wid = lax.axis_index("s") * num_cores + lax.axis_index("c")
        base = wid * rows_per_w
        pltpu.sync_copy(idx_hbm.at[pl.ds(wid * chunks_per_w, chunks_per_w)], idx_v)
        gathers = []
        for j in range(chunks_per_w):
            gathers.append(
                pltpu.async_copy(
                    table_hbm.at[idx_v.at[j]],
                    rows_v.at[pl.ds(j * _CHUNK, _CHUNK)],
                    gsems[j],
                )
            )
        outs = []
        for j in range(chunks_per_w):
            gathers[j].wait()
            outs.append(
                pltpu.async_copy(
                    rows_v.at[pl.ds(j * _CHUNK, _CHUNK)],
                    out_hbm.at[pl.ds(base + j * _CHUNK, _CHUNK)],
                    osem,
                )
            )
        for o in outs:
            o.wait()

    return gather_kernel


def _tc_body(emb_ref, wt_ref, out_ref):
    g = jax.lax.dot_general(
        emb_ref[...], wt_ref[...],
        dimension_numbers=(((1,), (1,)), ((), ())),
        preferred_element_type=jnp.float32,
    )
    gt = g.T
    m = jnp.max(gt, axis=0, keepdims=True)
    e = jnp.exp(gt - m)
    s = jnp.sum(e, axis=0, keepdims=True)
    out_ref[...] = e * (1.0 / s)


def _tc_body_acc(prev_ref, emb_ref, wt_ref, out_ref):
    del prev_ref
    _tc_body(emb_ref, wt_ref, out_ref)


def kernel(gating_input, table, W):
    batch = gating_input.shape[0]
    embed_dim = table.shape[1]
    n_experts = W.shape[1]

    info = plsc.get_sparse_core_info()
    n_workers = info.num_cores * info.num_subcores

    idx = gating_input.reshape(-1).astype(jnp.int32)
    idx2 = idx.reshape(batch // _CHUNK, _CHUNK)

    half = batch // 2
    hc = half // _CHUNK
    gather_half = _make_gather(half, embed_dim, n_workers, table.dtype)
    emb1 = gather_half(idx2[:hc], table)
    emb2 = gather_half(idx2[hc:], table)

    wt = W.T
    blk = 4096
    nh = half // blk

    out1 = pl.pallas_call(
        _tc_body,
        grid=(nh,),
        in_specs=[
            pl.BlockSpec((blk, embed_dim), lambda i: (i, 0)),
            pl.BlockSpec((n_experts, embed_dim), lambda i: (0, 0)),
        ],
        out_specs=pl.BlockSpec((n_experts, blk), lambda i: (0, i)),
        out_shape=jax.ShapeDtypeStruct((n_experts, batch), jnp.float32),
        compiler_params=pltpu.CompilerParams(
            dimension_semantics=("arbitrary",),
        ),
    )(emb1, wt)

    out2 = pl.pallas_call(
        _tc_body_acc,
        grid=(nh,),
        in_specs=[
            pl.BlockSpec(memory_space=pl.ANY),
            pl.BlockSpec((blk, embed_dim), lambda i: (i, 0)),
            pl.BlockSpec((n_experts, embed_dim), lambda i: (0, 0)),
        ],
        out_specs=pl.BlockSpec((n_experts, blk), lambda i: (0, i + nh)),
        out_shape=jax.ShapeDtypeStruct((n_experts, batch), jnp.float32),
        input_output_aliases={0: 0},
        compiler_params=pltpu.CompilerParams(
            dimension_semantics=("arbitrary",),
        ),
    )(out1, emb2, wt)
    return out2.T


# speedup vs baseline: 1.1145x; 1.1145x over previous
import functools

import jax
import jax.numpy as jnp
from jax import lax
from jax.experimental import pallas as pl
from jax.experimental.pallas import tpu as pltpu
from jax.experimental.pallas import tpu_sc as plsc

_EMBED_DIM = 128
_N_EXPERTS = 64

_CHUNK = 128


def _make_gather(n_rows, embed_dim, n_workers, dtype):
    rows_per_w = n_rows // n_workers
    chunks_per_w = rows_per_w // _CHUNK
    mesh = plsc.VectorSubcoreMesh(core_axis_name="c", subcore_axis_name="s")
    num_cores = plsc.get_sparse_core_info().num_cores

    @functools.partial(
        pl.kernel,
        mesh=mesh,
        out_type=jax.ShapeDtypeStruct((n_rows, embed_dim), dtype),
        scratch_types=[
            pltpu.VMEM((chunks_per_w, _CHUNK), jnp.int32),
            pltpu.VMEM((rows_per_w, embed_dim), dtype),
        ]
        + [pltpu.SemaphoreType.DMA] * chunks_per_w
        + [pltpu.SemaphoreType.DMA],
    )
    def gather_kernel(idx_hbm, table_hbm, out_hbm, idx_v, rows_v, *sems):
        gsems, osem = sems[:-1], sems[-1]
        wid = lax.axis_index("s") * num_cores + lax.axis_index("c")
        base = wid * rows_per_w
        pltpu.sync_copy(idx_hbm.at[pl.ds(wid * chunks_per_w, chunks_per_w)], idx_v)
        gathers = []
        for j in range(chunks_per_w):
            gathers.append(
                pltpu.async_copy(
                    table_hbm.at[idx_v.at[j]],
                    rows_v.at[pl.ds(j * _CHUNK, _CHUNK)],
                    gsems[j],
                )
            )
        outs = []
        for j in range(chunks_per_w):
            gathers[j].wait()
            outs.append(
                pltpu.async_copy(
                    rows_v.at[pl.ds(j * _CHUNK, _CHUNK)],
                    out_hbm.at[pl.ds(base + j * _CHUNK, _CHUNK)],
                    osem,
                )
            )
        for o in outs:
            o.wait()

    return gather_kernel


def _tc_body(emb_ref, wt_ref, out_ref):
    g = jax.lax.dot_general(
        emb_ref[...], wt_ref[...],
        dimension_numbers=(((1,), (1,)), ((), ())),
        preferred_element_type=jnp.float32,
    )
    gt = g.T
    m = jnp.max(gt, axis=0, keepdims=True)
    e = jnp.exp(gt - m)
    s = jnp.sum(e, axis=0, keepdims=True)
    out_ref[...] = e * (1.0 / s)


def kernel(gating_input, table, W):
    batch = gating_input.shape[0]
    embed_dim = table.shape[1]
    n_experts = W.shape[1]

    info = plsc.get_sparse_core_info()
    n_workers = info.num_cores * info.num_subcores

    idx = gating_input.reshape(-1).astype(jnp.int32)
    idx2 = idx.reshape(batch // _CHUNK, _CHUNK)

    emb = _make_gather(batch, embed_dim, n_workers, table.dtype)(idx2, table)

    blk = 8192
    out_t = pl.pallas_call(
        _tc_body,
        grid=(batch // blk,),
        in_specs=[
            pl.BlockSpec((blk, embed_dim), lambda i: (i, 0)),
            pl.BlockSpec((n_experts, embed_dim), lambda i: (0, 0)),
        ],
        out_specs=pl.BlockSpec((n_experts, blk), lambda i: (0, i)),
        out_shape=jax.ShapeDtypeStruct((n_experts, batch), jnp.float32),
        compiler_params=pltpu.CompilerParams(
            dimension_semantics=("arbitrary",),
        ),
    )(emb, W.T)
    return out_t.T
